# baseline (speedup 1.0000x reference)
import functools

import jax
import jax.numpy as jnp
from jax import lax
from jax.experimental import pallas as pl
from jax.experimental.pallas import tpu as pltpu
from jax.experimental.pallas import tpu_sc as plsc

H = 64
HP = 128
N = 2048
E = 8192
B = 64
NG = 3
NT = NG * N
ET = NG * E
NC = 2
NS = 16
NW = NC * NS
EPW = ET // NW
CH = 128
NCH = EPW // CH
HN = NT // 2
EPT = ET // NS
NCH2 = EPT // CH
HSLAB = HN // NS


def _sc_gather_body(h_hbm, src_hbm, out_hbm, idx_v, rows_v, sem, sem2):
    wid = lax.axis_index("s") * NC + lax.axis_index("c")
    base = pl.multiple_of(wid * EPW, EPW)
    pltpu.sync_copy(src_hbm.at[pl.ds(base, EPW)], idx_v)
    gathers = []
    for j in range(NCH):
        gathers.append(pltpu.async_copy(
            h_hbm.at[idx_v.at[pl.ds(j * CH, CH)]],
            rows_v.at[pl.ds(j * CH, CH)], sem))
    outs = []
    for j in range(NCH):
        gathers[j].wait()
        outs.append(pltpu.async_copy(
            rows_v.at[pl.ds(j * CH, CH)],
            out_hbm.at[pl.ds(base + j * CH, CH)], sem2))
    for cp in outs:
        cp.wait()


def _sc_scatter_body(msg_hbm, dstc_hbm, zeros_hbm, out_hbm,
                     idx_v, msg_v, acc_sh, sem1, sem2):
    c = lax.axis_index("c")
    s = lax.axis_index("s")
    slab = pl.multiple_of(s * HSLAB, 8)
    pltpu.sync_copy(
        dstc_hbm.at[pl.ds(pl.multiple_of(c * (NS * 16) + s * 16, 16), 16)],
        idx_v)
    pltpu.sync_copy(zeros_hbm, acc_sh.at[pl.ds(slab, HSLAB)])
    plsc.subcore_barrier()
    base = pl.multiple_of(s * EPT, EPT)
    stages = {}
    scats = []
    nsw = 0
    for j in range(NCH):
        stages[j] = pltpu.async_copy(
            msg_hbm.at[pl.ds(base + j * CH, CH)],
            msg_v.at[pl.ds(j * CH, CH)], sem1)
    for j in range(NCH2):
        sl = pl.ds((j % NCH) * CH, CH)
        stages[j].wait()
        scats.append(pltpu.async_copy(
            msg_v.at[sl], acc_sh.at[idx_v.at[j]], sem2, add=True))
        jn = j + NCH
        if jn < NCH2:
            scats[nsw].wait()
            nsw += 1
            stages[jn] = pltpu.async_copy(
                msg_hbm.at[pl.ds(base + jn * CH, CH)],
                msg_v.at[pl.ds((jn % NCH) * CH, CH)], sem1)
    for j in range(nsw, NCH2):
        scats[j].wait()
    plsc.subcore_barrier()
    pltpu.sync_copy(acc_sh.at[pl.ds(slab, HSLAB)], msg_v.at[pl.ds(0, HSLAB)])
    pltpu.sync_copy(
        msg_v.at[pl.ds(0, HSLAB)],
        out_hbm.at[pl.ds(pl.multiple_of(c * HN + s * HSLAB, 8), HSLAB)])


@functools.lru_cache(maxsize=1)
def _sc_kernels():
    mesh = plsc.VectorSubcoreMesh(
        core_axis_name="c", subcore_axis_name="s",
        num_cores=NC, num_subcores=NS)
    gather = pl.kernel(
        _sc_gather_body,
        out_type=jax.ShapeDtypeStruct((ET, HP), jnp.float32),
        mesh=mesh,
        compiler_params=pltpu.CompilerParams(use_tc_tiling_on_sc=True),
        scratch_types=[
            pltpu.VMEM((EPW,), jnp.int32),
            pltpu.VMEM((EPW, HP), jnp.float32),
            pltpu.SemaphoreType.DMA,
            pltpu.SemaphoreType.DMA,
        ],
    )
    scatter = pl.kernel(
        _sc_scatter_body,
        out_type=jax.ShapeDtypeStruct((NT, HP), jnp.float32),
        mesh=mesh,
        compiler_params=pltpu.CompilerParams(use_tc_tiling_on_sc=True),
        scratch_types=[
            pltpu.VMEM((16, CH), jnp.int32),
            pltpu.VMEM((EPW, HP), jnp.float32),
            pltpu.VMEM_SHARED((HN + 8, HP), jnp.float32),
            pltpu.SemaphoreType.DMA,
            pltpu.SemaphoreType.DMA,
        ],
    )
    return gather, scatter


def _sc_gather(h, src):
    return _sc_kernels()[0](h, src)


def _sc_scatter(msg, dstg, zeros):
    return _sc_kernels()[1](msg, dstg, zeros)


def _proj_body(x2_ref, xp_ref, w_ref, b_ref, o_ref):
    w = w_ref[...].astype(jnp.bfloat16)
    b = b_ref[...]
    z = jnp.zeros((N, H), jnp.float32)
    for g in range(NG):
        x = xp_ref[...] if g == 2 else x2_ref[g]
        hg = jnp.maximum(
            jnp.dot(x.astype(jnp.bfloat16), w,
                    preferred_element_type=jnp.float32) + b, 0.0)
        o_ref[g * N:(g + 1) * N, :] = jnp.concatenate([hg, z], axis=1)


_MSG_BLK = 2048


def _msg_body(hs_ref, ef_ref, rmat_ref, w2p_ref, o_ref):
    hs16 = hs_ref[:, 0:H].astype(jnp.bfloat16)
    efb = jnp.dot(ef_ref[...].astype(jnp.bfloat16), rmat_ref[...],
                  preferred_element_type=jnp.float32).astype(jnp.bfloat16)
    hst = jnp.concatenate([hs16] * 16, axis=1)
    prod = (efb.astype(jnp.float32) * hst.astype(jnp.float32)
            ).astype(jnp.bfloat16)
    u = jnp.concatenate([prod, hs16], axis=1)
    msg = jnp.dot(u, w2p_ref[...], preferred_element_type=jnp.float32)
    o_ref[...] = jnp.concatenate(
        [msg, jnp.zeros((_MSG_BLK, HP - H), jnp.float32)], axis=1)


def _nt_dot(a, b):
    return lax.dot_general(a.astype(jnp.bfloat16), b.astype(jnp.bfloat16),
                           (((1,), (1,)), ((), ())),
                           preferred_element_type=jnp.float32)


def _nn_dot(a, b):
    return jnp.dot(a.astype(jnp.bfloat16), b.astype(jnp.bfloat16),
                   preferred_element_type=jnp.float32)


def _gru_body(aggp_ref, h_ref, wih_ref, whh_ref, bih_ref, bhh_ref,
              bconv_ref, o_ref):
    agg = aggp_ref[:, 0:H] + bconv_ref[...]
    m = jnp.maximum(agg, 0.0)
    h = h_ref[:, 0:H]
    gi = _nt_dot(m, wih_ref[...]) + bih_ref[...]
    gh = _nt_dot(h, whh_ref[...]) + bhh_ref[...]
    r = jax.nn.sigmoid(gi[:, 0:H] + gh[:, 0:H])
    z = jax.nn.sigmoid(gi[:, H:2 * H] + gh[:, H:2 * H])
    nn_ = jnp.tanh(gi[:, 2 * H:3 * H] + r * gh[:, 2 * H:3 * H])
    hn = (1.0 - z) * nn_ + z * h
    o_ref[...] = jnp.concatenate(
        [hn, jnp.zeros((NT, HP - H), jnp.float32)], axis=1)


def _prelu(x, a):
    return jnp.where(x >= 0.0, x, a * x)


def _s2s_body(h0_ref, h_ref, n2gc_ref, n2gr_ref, rinds_ref,
              wihs_ref, whhs_ref, bihs_ref, bhhs_ref,
              wsp_ref, bsp_ref, asp_ref, wem_ref, bem_ref, aem_ref,
              w1_ref, b1_ref, a1_ref, w2_ref, b2_ref, a2_ref,
              w3_ref, b3_ref, o_ref):
    M = (n2gc_ref[...] == lax.broadcasted_iota(jnp.int32, (N, B), 1)
         ).astype(jnp.float32)
    MT = (n2gr_ref[...] == lax.broadcasted_iota(jnp.int32, (B, N), 0)
          ).astype(jnp.float32)
    gs = []
    for g in range(NG):
        feat = jnp.concatenate(
            [h0_ref[g * N:(g + 1) * N, 0:H], h_ref[g * N:(g + 1) * N, 0:H]],
            axis=1)
        q_star = jnp.zeros((B, 4 * H), jnp.float32)
        hs = jnp.zeros((B, 2 * H), jnp.float32)
        cs = jnp.zeros((B, 2 * H), jnp.float32)
        for _ in range(3):
            gates = (_nt_dot(q_star, wihs_ref[...]) + bihs_ref[...]
                     + _nt_dot(hs, whhs_ref[...]) + bhhs_ref[...])
            gi_ = gates[:, 0:2 * H]
            gf_ = gates[:, 2 * H:4 * H]
            gg_ = gates[:, 4 * H:6 * H]
            go_ = gates[:, 6 * H:8 * H]
            cs = jax.nn.sigmoid(gf_) * cs + jax.nn.sigmoid(gi_) * jnp.tanh(gg_)
            hs = jax.nn.sigmoid(go_) * jnp.tanh(cs)
            q = hs
            mq = _nn_dot(M, q)
            e = jnp.sum(feat * mq, axis=1, keepdims=True)
            smat = jnp.where(M > 0.0, e, -1e30)
            emax = jnp.max(smat, axis=0, keepdims=True)
            emax_n = jnp.sum(M * emax, axis=1, keepdims=True)
            ee = jnp.exp(e - emax_n)
            denom = jnp.sum(M * ee, axis=0, keepdims=True)
            denom_n = jnp.sum(M * denom, axis=1, keepdims=True)
            alpha = ee / denom_n
            readout = _nn_dot(MT, alpha * feat)
            q_star = jnp.concatenate([q, readout], axis=1)
        gsp = _nn_dot(q_star, wsp_ref[...]) + bsp_ref[...]
        gs.append(_prelu(gsp, asp_ref[...]))
    r_graph = 0.5 * (gs[0] + gs[1])
    cat = jnp.concatenate([r_graph, gs[2]], axis=1)
    re = _prelu(_nn_dot(cat, wem_ref[...]) + bem_ref[...], aem_ref[...])
    for r_i in range(2):
        ind = jnp.broadcast_to(rinds_ref[r_i:r_i + 1, :], (B, 8))
        inp = jnp.concatenate([gs[r_i] - re, ind], axis=1)
        h1 = _prelu(_nn_dot(inp, w1_ref[...]) + b1_ref[...], a1_ref[...])
        h2 = _prelu(_nn_dot(h1, w2_ref[...]) + b2_ref[...], a2_ref[...])
        o_ref[r_i, :, :] = _nn_dot(h2, w3_ref[...]) + b3_ref[...]


def _proj_call(x2, xp, w, b):
    return pl.pallas_call(
        _proj_body,
        out_shape=jax.ShapeDtypeStruct((NT, HP), jnp.float32),
    )(x2, xp, w, b)


def _msg_call(hs, ef, rmat, w2p):
    grid = ET // _MSG_BLK
    return pl.pallas_call(
        _msg_body,
        grid=(grid,),
        in_specs=[
            pl.BlockSpec((_MSG_BLK, HP), lambda i: (i, 0)),
            pl.BlockSpec((_MSG_BLK, 16), lambda i: (i, 0)),
            pl.BlockSpec((16, 16 * H), lambda i: (0, 0)),
            pl.BlockSpec((17 * H, H), lambda i: (0, 0)),
        ],
        out_specs=pl.BlockSpec((_MSG_BLK, HP), lambda i: (i, 0)),
        out_shape=jax.ShapeDtypeStruct((ET, HP), jnp.float32),
    )(hs, ef, rmat, w2p)


def _gru_call(aggp, h, wih, whh, bih, bhh, bconv):
    return pl.pallas_call(
        _gru_body,
        out_shape=jax.ShapeDtypeStruct((NT, HP), jnp.float32),
    )(aggp, h, wih, whh, bih, bhh, bconv)


def _s2s_call(h0, h, n2gc, n2gr, rinds, args):
    return pl.pallas_call(
        _s2s_body,
        out_shape=jax.ShapeDtypeStruct((2, B, 128), jnp.float32),
    )(h0, h, n2gc, n2gr, rinds, *args)


def kernel(r_node_feats, r_edge_feats, p_node_feats, p_edge_feats, rinds,
           params, r_edge_index, p_edge_index, node2graph):
    p = params
    f32 = jnp.float32

    ef = jnp.concatenate([r_edge_feats[0], r_edge_feats[1], p_edge_feats],
                         axis=0)
    src = jnp.concatenate([
        r_edge_index[0, 0], r_edge_index[1, 0] + N, p_edge_index[0] + 2 * N])
    dst = jnp.concatenate([
        r_edge_index[0, 1], r_edge_index[1, 1] + N, p_edge_index[1] + 2 * N])
    dst0 = jnp.where(dst < HN, dst, HN)
    dst1 = jnp.where(dst >= HN, dst - HN, HN)
    dstc = jnp.concatenate([
        jnp.pad(d.reshape(NS, NCH2, CH), ((0, 0), (0, 16 - NCH2), (0, 0)))
        for d in (dst0, dst1)], axis=0).reshape(NC * NS * 16, CH)
    zeros = jnp.zeros((HSLAB, HP), f32)

    bf16 = jnp.bfloat16
    rmat = jnp.kron(jnp.eye(16, dtype=f32), jnp.ones((1, H), f32)).astype(bf16)
    w2p = jnp.concatenate([
        p['W_edge'].reshape(16 * H, H),
        p['b_edge'].reshape(H, H)], axis=0).astype(bf16)

    h0 = _proj_call(r_node_feats, p_node_feats,
                    p['W_proj'], p['b_proj'].reshape(1, H))

    def _step(_, h):
        hs = _sc_gather(h, src)
        msg = _msg_call(hs, ef, rmat, w2p)
        aggp = _sc_scatter(msg, dstc, zeros)
        return _gru_call(aggp, h, p['W_ih'], p['W_hh'],
                         p['b_ih'].reshape(1, 3 * H),
                         p['b_hh'].reshape(1, 3 * H),
                         p['b_conv'].reshape(1, H))

    h = lax.fori_loop(0, 3, _step, h0)

    s2s_args = (
        p['W_ih_s'], p['W_hh_s'],
        p['b_ih_s'].reshape(1, 8 * H), p['b_hh_s'].reshape(1, 8 * H),
        p['W_sp'], p['b_sp'].reshape(1, 1024), p['a_sp'].reshape(1, 1),
        p['W_em'], p['b_em'].reshape(1, 1024), p['a_em'].reshape(1, 1),
        p['W1'], p['b1'].reshape(1, 512), p['a1'].reshape(1, 1),
        p['W2'], p['b2'].reshape(1, 512), p['a2'].reshape(1, 1),
        jnp.pad(p['W3'], ((0, 0), (0, 126))),
        jnp.pad(p['b3'], (0, 126)).reshape(1, 128),
    )
    res = _s2s_call(h0, h, node2graph.reshape(N, 1), node2graph.reshape(1, N),
                    rinds, s2s_args)
    return jnp.transpose(res[:, :, 0:2], (1, 0, 2))


# speedup vs baseline: 1.0018x; 1.0018x over previous
import functools

import jax
import jax.numpy as jnp
from jax import lax
from jax.experimental import pallas as pl
from jax.experimental.pallas import tpu as pltpu
from jax.experimental.pallas import tpu_sc as plsc

H = 64
HP = 128
N = 2048
E = 8192
B = 64
NG = 3
NT = NG * N
ET = NG * E
NC = 2
NS = 16
NW = NC * NS
EPW = ET // NW
CH = 128
NCH = EPW // CH
HN = NT // 2
EPT = ET // NS
NCH2 = EPT // CH
HSLAB = HN // NS


def _sc_gather_body(h_hbm, src_hbm, out_hbm, idx_v, rows_v, sem, sem2):
    wid = lax.axis_index("s") * NC + lax.axis_index("c")
    base = pl.multiple_of(wid * EPW, EPW)
    pltpu.sync_copy(src_hbm.at[pl.ds(base, EPW)], idx_v)
    gathers = []
    for j in range(NCH):
        gathers.append(pltpu.async_copy(
            h_hbm.at[idx_v.at[pl.ds(j * CH, CH)]],
            rows_v.at[pl.ds(j * CH, CH)], sem))
    outs = []
    for j in range(NCH):
        gathers[j].wait()
        outs.append(pltpu.async_copy(
            rows_v.at[pl.ds(j * CH, CH)],
            out_hbm.at[pl.ds(base + j * CH, CH)], sem2))
    for cp in outs:
        cp.wait()


def _sc_scatter_body(msg_hbm, dstc_hbm, zeros_hbm, out_hbm,
                     idx_v, msg_v, acc_sh, sem1, sem2):
    c = lax.axis_index("c")
    s = lax.axis_index("s")
    slab = pl.multiple_of(s * HSLAB, 8)
    pltpu.sync_copy(
        dstc_hbm.at[pl.ds(pl.multiple_of(c * (NS * 16) + s * 16, 16), 16)],
        idx_v)
    pltpu.sync_copy(zeros_hbm, acc_sh.at[pl.ds(slab, HSLAB)])
    plsc.subcore_barrier()
    base = pl.multiple_of(s * EPT, EPT)
    stages = {}
    scats = []
    nsw = 0
    for j in range(NCH):
        stages[j] = pltpu.async_copy(
            msg_hbm.at[pl.ds(base + j * CH, CH)],
            msg_v.at[pl.ds(j * CH, CH)], sem1)
    for j in range(NCH2):
        sl = pl.ds((j % NCH) * CH, CH)
        stages[j].wait()
        scats.append(pltpu.async_copy(
            msg_v.at[sl], acc_sh.at[idx_v.at[j]], sem2, add=True))
        jn = j + NCH
        if jn < NCH2:
            scats[nsw].wait()
            nsw += 1
            stages[jn] = pltpu.async_copy(
                msg_hbm.at[pl.ds(base + jn * CH, CH)],
                msg_v.at[pl.ds((jn % NCH) * CH, CH)], sem1)
    for j in range(nsw, NCH2):
        scats[j].wait()
    plsc.subcore_barrier()
    pltpu.sync_copy(acc_sh.at[pl.ds(slab, HSLAB)], msg_v.at[pl.ds(0, HSLAB)])
    pltpu.sync_copy(
        msg_v.at[pl.ds(0, HSLAB)],
        out_hbm.at[pl.ds(pl.multiple_of(c * HN + s * HSLAB, 8), HSLAB)])


@functools.lru_cache(maxsize=1)
def _sc_kernels():
    mesh = plsc.VectorSubcoreMesh(
        core_axis_name="c", subcore_axis_name="s",
        num_cores=NC, num_subcores=NS)
    gather = pl.kernel(
        _sc_gather_body,
        out_type=jax.ShapeDtypeStruct((ET, HP), jnp.float32),
        mesh=mesh,
        compiler_params=pltpu.CompilerParams(use_tc_tiling_on_sc=True),
        scratch_types=[
            pltpu.VMEM((EPW,), jnp.int32),
            pltpu.VMEM((EPW, HP), jnp.float32),
            pltpu.SemaphoreType.DMA,
            pltpu.SemaphoreType.DMA,
        ],
    )
    scatter = pl.kernel(
        _sc_scatter_body,
        out_type=jax.ShapeDtypeStruct((NT, HP), jnp.float32),
        mesh=mesh,
        compiler_params=pltpu.CompilerParams(use_tc_tiling_on_sc=True),
        scratch_types=[
            pltpu.VMEM((16, CH), jnp.int32),
            pltpu.VMEM((EPW, HP), jnp.float32),
            pltpu.VMEM_SHARED((HN + 8, HP), jnp.float32),
            pltpu.SemaphoreType.DMA,
            pltpu.SemaphoreType.DMA,
        ],
    )
    return gather, scatter


def _sc_gather(h, src):
    return _sc_kernels()[0](h, src)


def _sc_scatter(msg, dstg, zeros):
    return _sc_kernels()[1](msg, dstg, zeros)


def _proj_body(x2_ref, xp_ref, w_ref, b_ref, o_ref):
    w = w_ref[...]
    b = b_ref[...]
    z = jnp.zeros((N, H), jnp.float32)
    for g in range(NG):
        x = xp_ref[...] if g == 2 else x2_ref[g]
        hg = jnp.maximum(
            jnp.dot(x, w, preferred_element_type=jnp.float32) + b, 0.0)
        o_ref[g * N:(g + 1) * N, :] = jnp.concatenate([hg, z], axis=1)


_MSG_BLK = 2048


def _msg_body(hs_ref, ef_ref, rmat_ref, w2p_ref, o_ref):
    hs16 = hs_ref[:, 0:H].astype(jnp.bfloat16)
    efb = jnp.dot(ef_ref[...].astype(jnp.bfloat16), rmat_ref[...],
                  preferred_element_type=jnp.float32).astype(jnp.bfloat16)
    hst = jnp.concatenate([hs16] * 16, axis=1)
    prod = (efb.astype(jnp.float32) * hst.astype(jnp.float32)
            ).astype(jnp.bfloat16)
    u = jnp.concatenate([prod, hs16], axis=1)
    msg = jnp.dot(u, w2p_ref[...], preferred_element_type=jnp.float32)
    o_ref[...] = jnp.concatenate(
        [msg, jnp.zeros((_MSG_BLK, HP - H), jnp.float32)], axis=1)


def _nt_dot(a, b):
    return lax.dot_general(a, b, (((1,), (1,)), ((), ())),
                           preferred_element_type=jnp.float32)


def _nn_dot(a, b):
    return jnp.dot(a, b, preferred_element_type=jnp.float32)


def _gru_body(aggp_ref, h_ref, wih_ref, whh_ref, bih_ref, bhh_ref,
              bconv_ref, o_ref):
    agg = aggp_ref[:, 0:H] + bconv_ref[...]
    m = jnp.maximum(agg, 0.0)
    h = h_ref[:, 0:H]
    gi = _nt_dot(m, wih_ref[...]) + bih_ref[...]
    gh = _nt_dot(h, whh_ref[...]) + bhh_ref[...]
    r = jax.nn.sigmoid(gi[:, 0:H] + gh[:, 0:H])
    z = jax.nn.sigmoid(gi[:, H:2 * H] + gh[:, H:2 * H])
    nn_ = jnp.tanh(gi[:, 2 * H:3 * H] + r * gh[:, 2 * H:3 * H])
    hn = (1.0 - z) * nn_ + z * h
    o_ref[...] = jnp.concatenate(
        [hn, jnp.zeros((NT, HP - H), jnp.float32)], axis=1)


def _prelu(x, a):
    return jnp.where(x >= 0.0, x, a * x)


def _s2s_body(h0_ref, h_ref, n2gc_ref, n2gr_ref, rinds_ref,
              wihs_ref, whhs_ref, bihs_ref, bhhs_ref,
              wsp_ref, bsp_ref, asp_ref, wem_ref, bem_ref, aem_ref,
              w1_ref, b1_ref, a1_ref, w2_ref, b2_ref, a2_ref,
              w3_ref, b3_ref, o_ref):
    M = (n2gc_ref[...] == lax.broadcasted_iota(jnp.int32, (N, B), 1)
         ).astype(jnp.float32)
    MT = (n2gr_ref[...] == lax.broadcasted_iota(jnp.int32, (B, N), 0)
          ).astype(jnp.float32)
    gs = []
    for g in range(NG):
        feat = jnp.concatenate(
            [h0_ref[g * N:(g + 1) * N, 0:H], h_ref[g * N:(g + 1) * N, 0:H]],
            axis=1)
        q_star = jnp.zeros((B, 4 * H), jnp.float32)
        hs = jnp.zeros((B, 2 * H), jnp.float32)
        cs = jnp.zeros((B, 2 * H), jnp.float32)
        for _ in range(3):
            gates = (_nt_dot(q_star, wihs_ref[...]) + bihs_ref[...]
                     + _nt_dot(hs, whhs_ref[...]) + bhhs_ref[...])
            gi_ = gates[:, 0:2 * H]
            gf_ = gates[:, 2 * H:4 * H]
            gg_ = gates[:, 4 * H:6 * H]
            go_ = gates[:, 6 * H:8 * H]
            cs = jax.nn.sigmoid(gf_) * cs + jax.nn.sigmoid(gi_) * jnp.tanh(gg_)
            hs = jax.nn.sigmoid(go_) * jnp.tanh(cs)
            q = hs
            mq = _nn_dot(M, q)
            e = jnp.sum(feat * mq, axis=1, keepdims=True)
            smat = jnp.where(M > 0.0, e, -1e30)
            emax = jnp.max(smat, axis=0, keepdims=True)
            emax_n = jnp.sum(M * emax, axis=1, keepdims=True)
            ee = jnp.exp(e - emax_n)
            denom = jnp.sum(M * ee, axis=0, keepdims=True)
            denom_n = jnp.sum(M * denom, axis=1, keepdims=True)
            alpha = ee / denom_n
            readout = _nn_dot(MT, alpha * feat)
            q_star = jnp.concatenate([q, readout], axis=1)
        gsp = _nn_dot(q_star, wsp_ref[...]) + bsp_ref[...]
        gs.append(_prelu(gsp, asp_ref[...]))
    r_graph = 0.5 * (gs[0] + gs[1])
    cat = jnp.concatenate([r_graph, gs[2]], axis=1)
    re = _prelu(_nn_dot(cat, wem_ref[...]) + bem_ref[...], aem_ref[...])
    for r_i in range(2):
        ind = jnp.broadcast_to(rinds_ref[r_i:r_i + 1, :], (B, 8))
        inp = jnp.concatenate([gs[r_i] - re, ind], axis=1)
        h1 = _prelu(_nn_dot(inp, w1_ref[...]) + b1_ref[...], a1_ref[...])
        h2 = _prelu(_nn_dot(h1, w2_ref[...]) + b2_ref[...], a2_ref[...])
        o_ref[r_i, :, :] = _nn_dot(h2, w3_ref[...]) + b3_ref[...]


def _proj_call(x2, xp, w, b):
    return pl.pallas_call(
        _proj_body,
        out_shape=jax.ShapeDtypeStruct((NT, HP), jnp.float32),
    )(x2, xp, w, b)


def _msg_call(hs, ef, rmat, w2p):
    grid = ET // _MSG_BLK
    return pl.pallas_call(
        _msg_body,
        grid=(grid,),
        in_specs=[
            pl.BlockSpec((_MSG_BLK, HP), lambda i: (i, 0)),
            pl.BlockSpec((_MSG_BLK, 16), lambda i: (i, 0)),
            pl.BlockSpec((16, 16 * H), lambda i: (0, 0)),
            pl.BlockSpec((17 * H, H), lambda i: (0, 0)),
        ],
        out_specs=pl.BlockSpec((_MSG_BLK, HP), lambda i: (i, 0)),
        out_shape=jax.ShapeDtypeStruct((ET, HP), jnp.float32),
    )(hs, ef, rmat, w2p)


def _gru_call(aggp, h, wih, whh, bih, bhh, bconv):
    return pl.pallas_call(
        _gru_body,
        out_shape=jax.ShapeDtypeStruct((NT, HP), jnp.float32),
    )(aggp, h, wih, whh, bih, bhh, bconv)


def _s2s_call(h0, h, n2gc, n2gr, rinds, args):
    return pl.pallas_call(
        _s2s_body,
        out_shape=jax.ShapeDtypeStruct((2, B, 128), jnp.float32),
    )(h0, h, n2gc, n2gr, rinds, *args)


def kernel(r_node_feats, r_edge_feats, p_node_feats, p_edge_feats, rinds,
           params, r_edge_index, p_edge_index, node2graph):
    p = params
    f32 = jnp.float32

    ef = jnp.concatenate([r_edge_feats[0], r_edge_feats[1], p_edge_feats],
                         axis=0)
    src = jnp.concatenate([
        r_edge_index[0, 0], r_edge_index[1, 0] + N, p_edge_index[0] + 2 * N])
    dst = jnp.concatenate([
        r_edge_index[0, 1], r_edge_index[1, 1] + N, p_edge_index[1] + 2 * N])
    dst0 = jnp.where(dst < HN, dst, HN)
    dst1 = jnp.where(dst >= HN, dst - HN, HN)
    dstc = jnp.concatenate([
        jnp.pad(d.reshape(NS, NCH2, CH), ((0, 0), (0, 16 - NCH2), (0, 0)))
        for d in (dst0, dst1)], axis=0).reshape(NC * NS * 16, CH)
    zeros = jnp.zeros((HSLAB, HP), f32)

    bf16 = jnp.bfloat16
    rmat = jnp.kron(jnp.eye(16, dtype=f32), jnp.ones((1, H), f32)).astype(bf16)
    w2p = jnp.concatenate([
        p['W_edge'].reshape(16 * H, H),
        p['b_edge'].reshape(H, H)], axis=0).astype(bf16)

    h0 = _proj_call(r_node_feats, p_node_feats,
                    p['W_proj'], p['b_proj'].reshape(1, H))

    def _step(_, h):
        hs = _sc_gather(h, src)
        msg = _msg_call(hs, ef, rmat, w2p)
        aggp = _sc_scatter(msg, dstc, zeros)
        return _gru_call(aggp, h, p['W_ih'], p['W_hh'],
                         p['b_ih'].reshape(1, 3 * H),
                         p['b_hh'].reshape(1, 3 * H),
                         p['b_conv'].reshape(1, H))

    h = lax.fori_loop(0, 3, _step, h0)

    s2s_args = (
        p['W_ih_s'], p['W_hh_s'],
        p['b_ih_s'].reshape(1, 8 * H), p['b_hh_s'].reshape(1, 8 * H),
        p['W_sp'], p['b_sp'].reshape(1, 1024), p['a_sp'].reshape(1, 1),
        p['W_em'], p['b_em'].reshape(1, 1024), p['a_em'].reshape(1, 1),
        p['W1'], p['b1'].reshape(1, 512), p['a1'].reshape(1, 1),
        p['W2'], p['b2'].reshape(1, 512), p['a2'].reshape(1, 1),
        jnp.pad(p['W3'], ((0, 0), (0, 126))),
        jnp.pad(p['b3'], (0, 126)).reshape(1, 128),
    )
    res = _s2s_call(h0, h, node2graph.reshape(N, 1), node2graph.reshape(1, N),
                    rinds, s2s_args)
    return jnp.transpose(res[:, :, 0:2], (1, 0, 2))


# speedup vs baseline: 1.0060x; 1.0042x over previous
import functools

import jax
import jax.numpy as jnp
from jax import lax
from jax.experimental import pallas as pl
from jax.experimental.pallas import tpu as pltpu
from jax.experimental.pallas import tpu_sc as plsc

H = 64
HP = 128
N = 2048
E = 8192
B = 64
NG = 3
NT = NG * N
ET = NG * E
NC = 2
NS = 16
NW = NC * NS
EPW = ET // NW
CH = 128
NCH = EPW // CH
HN = NT // 2
EPT = ET // NS
NCH2 = EPT // CH
HSLAB = HN // NS


def _sc_gather_body(h_hbm, src_hbm, out_hbm, idx_v, rows_v, sem, sem2):
    wid = lax.axis_index("s") * NC + lax.axis_index("c")
    base = pl.multiple_of(wid * EPW, EPW)
    pltpu.sync_copy(src_hbm.at[pl.ds(base, EPW)], idx_v)
    gathers = []
    for j in range(NCH):
        gathers.append(pltpu.async_copy(
            h_hbm.at[idx_v.at[pl.ds(j * CH, CH)]],
            rows_v.at[pl.ds(j * CH, CH)], sem))
    outs = []
    for j in range(NCH):
        gathers[j].wait()
        outs.append(pltpu.async_copy(
            rows_v.at[pl.ds(j * CH, CH)],
            out_hbm.at[pl.ds(base + j * CH, CH)], sem2))
    for cp in outs:
        cp.wait()


def _sc_scatter_body(msg_hbm, dstc_hbm, zeros_hbm, out_hbm,
                     idx_v, msg_v, acc_sh, sem1, sem2):
    c = lax.axis_index("c")
    s = lax.axis_index("s")
    slab = pl.multiple_of(s * HSLAB, 8)
    pltpu.sync_copy(
        dstc_hbm.at[pl.ds(pl.multiple_of(c * (NS * 16) + s * 16, 16), 16)],
        idx_v)
    pltpu.sync_copy(zeros_hbm, acc_sh.at[pl.ds(slab, HSLAB)])
    plsc.subcore_barrier()
    base = pl.multiple_of(s * EPT, EPT)
    stages = {}
    scats = []
    nsw = 0
    for j in range(NCH):
        stages[j] = pltpu.async_copy(
            msg_hbm.at[pl.ds(base + j * CH, CH)],
            msg_v.at[pl.ds(j * CH, CH)], sem1)
    for j in range(NCH2):
        sl = pl.ds((j % NCH) * CH, CH)
        stages[j].wait()
        scats.append(pltpu.async_copy(
            msg_v.at[sl], acc_sh.at[idx_v.at[j]], sem2, add=True))
        jn = j + NCH
        if jn < NCH2:
            scats[nsw].wait()
            nsw += 1
            stages[jn] = pltpu.async_copy(
                msg_hbm.at[pl.ds(base + jn * CH, CH)],
                msg_v.at[pl.ds((jn % NCH) * CH, CH)], sem1)
    for j in range(nsw, NCH2):
        scats[j].wait()
    plsc.subcore_barrier()
    pltpu.sync_copy(acc_sh.at[pl.ds(slab, HSLAB)], msg_v.at[pl.ds(0, HSLAB)])
    pltpu.sync_copy(
        msg_v.at[pl.ds(0, HSLAB)],
        out_hbm.at[pl.ds(pl.multiple_of(c * HN + s * HSLAB, 8), HSLAB)])


@functools.lru_cache(maxsize=1)
def _sc_kernels():
    mesh = plsc.VectorSubcoreMesh(
        core_axis_name="c", subcore_axis_name="s",
        num_cores=NC, num_subcores=NS)
    gather = pl.kernel(
        _sc_gather_body,
        out_type=jax.ShapeDtypeStruct((ET, HP), jnp.float32),
        mesh=mesh,
        compiler_params=pltpu.CompilerParams(use_tc_tiling_on_sc=True),
        scratch_types=[
            pltpu.VMEM((EPW,), jnp.int32),
            pltpu.VMEM((EPW, HP), jnp.float32),
            pltpu.SemaphoreType.DMA,
            pltpu.SemaphoreType.DMA,
        ],
    )
    scatter = pl.kernel(
        _sc_scatter_body,
        out_type=jax.ShapeDtypeStruct((NT, HP), jnp.float32),
        mesh=mesh,
        compiler_params=pltpu.CompilerParams(use_tc_tiling_on_sc=True),
        scratch_types=[
            pltpu.VMEM((16, CH), jnp.int32),
            pltpu.VMEM((EPW, HP), jnp.float32),
            pltpu.VMEM_SHARED((HN + 8, HP), jnp.float32),
            pltpu.SemaphoreType.DMA,
            pltpu.SemaphoreType.DMA,
        ],
    )
    return gather, scatter


def _sc_gather(h, src):
    return _sc_kernels()[0](h, src)


def _sc_scatter(msg, dstg, zeros):
    return _sc_kernels()[1](msg, dstg, zeros)


def _proj_body(x2_ref, xp_ref, w_ref, b_ref, o_ref):
    w = w_ref[...]
    b = b_ref[...]
    z = jnp.zeros((N, H), jnp.float32)
    for g in range(NG):
        x = xp_ref[...] if g == 2 else x2_ref[g]
        hg = jnp.maximum(
            jnp.dot(x, w, preferred_element_type=jnp.float32) + b, 0.0)
        o_ref[g * N:(g + 1) * N, :] = jnp.concatenate([hg, z], axis=1)


_MSG_BLK = 2048


def _msg_body(hs_ref, ef_ref, rmat_ref, whi_ref, o_ref):
    hs = hs_ref[:, 0:H]
    efb = jnp.dot(ef_ref[...], rmat_ref[...],
                  preferred_element_type=jnp.float32)
    hst = jnp.concatenate([hs] * 16, axis=1)
    u = jnp.concatenate([efb * hst, hs], axis=1)
    msg = jnp.dot(u, whi_ref[...], preferred_element_type=jnp.float32)
    o_ref[...] = jnp.concatenate(
        [msg, jnp.zeros((_MSG_BLK, HP - H), jnp.float32)], axis=1)


def _msg_call(hs, ef, rmat, whi):
    grid = ET // _MSG_BLK
    return pl.pallas_call(
        _msg_body,
        grid=(grid,),
        in_specs=[
            pl.BlockSpec((_MSG_BLK, HP), lambda i: (i, 0)),
            pl.BlockSpec((_MSG_BLK, 16), lambda i: (i, 0)),
            pl.BlockSpec((16, 16 * H), lambda i: (0, 0)),
            pl.BlockSpec((17 * H, H), lambda i: (0, 0)),
        ],
        out_specs=pl.BlockSpec((_MSG_BLK, HP), lambda i: (i, 0)),
        out_shape=jax.ShapeDtypeStruct((ET, HP), jnp.float32),
    )(hs, ef, rmat, whi)


def _nt_dot(a, b):
    return lax.dot_general(a, b, (((1,), (1,)), ((), ())),
                           preferred_element_type=jnp.float32)


def _nn_dot(a, b):
    return jnp.dot(a, b, preferred_element_type=jnp.float32)


def _gru_body(aggp_ref, h_ref, wih_ref, whh_ref, bih_ref, bhh_ref,
              bconv_ref, o_ref):
    agg = aggp_ref[:, 0:H] + bconv_ref[...]
    m = jnp.maximum(agg, 0.0)
    h = h_ref[:, 0:H]
    gi = _nt_dot(m, wih_ref[...]) + bih_ref[...]
    gh = _nt_dot(h, whh_ref[...]) + bhh_ref[...]
    r = jax.nn.sigmoid(gi[:, 0:H] + gh[:, 0:H])
    z = jax.nn.sigmoid(gi[:, H:2 * H] + gh[:, H:2 * H])
    nn_ = jnp.tanh(gi[:, 2 * H:3 * H] + r * gh[:, 2 * H:3 * H])
    hn = (1.0 - z) * nn_ + z * h
    o_ref[...] = jnp.concatenate(
        [hn, jnp.zeros((NT, HP - H), jnp.float32)], axis=1)


def _prelu(x, a):
    return jnp.where(x >= 0.0, x, a * x)


def _s2s_body(h0_ref, h_ref, n2gc_ref, n2gr_ref, rinds_ref,
              wihs_ref, whhs_ref, bihs_ref, bhhs_ref,
              wsp_ref, bsp_ref, asp_ref, wem_ref, bem_ref, aem_ref,
              w1_ref, b1_ref, a1_ref, w2_ref, b2_ref, a2_ref,
              w3_ref, b3_ref, o_ref):
    M = (n2gc_ref[...] == lax.broadcasted_iota(jnp.int32, (N, B), 1)
         ).astype(jnp.float32)
    MT = (n2gr_ref[...] == lax.broadcasted_iota(jnp.int32, (B, N), 0)
          ).astype(jnp.float32)
    gs = []
    for g in range(NG):
        feat = jnp.concatenate(
            [h0_ref[g * N:(g + 1) * N, 0:H], h_ref[g * N:(g + 1) * N, 0:H]],
            axis=1)
        q_star = jnp.zeros((B, 4 * H), jnp.float32)
        hs = jnp.zeros((B, 2 * H), jnp.float32)
        cs = jnp.zeros((B, 2 * H), jnp.float32)
        for _ in range(3):
            gates = (_nt_dot(q_star, wihs_ref[...]) + bihs_ref[...]
                     + _nt_dot(hs, whhs_ref[...]) + bhhs_ref[...])
            gi_ = gates[:, 0:2 * H]
            gf_ = gates[:, 2 * H:4 * H]
            gg_ = gates[:, 4 * H:6 * H]
            go_ = gates[:, 6 * H:8 * H]
            cs = jax.nn.sigmoid(gf_) * cs + jax.nn.sigmoid(gi_) * jnp.tanh(gg_)
            hs = jax.nn.sigmoid(go_) * jnp.tanh(cs)
            q = hs
            mq = _nn_dot(M, q)
            e = jnp.sum(feat * mq, axis=1, keepdims=True)
            smat = jnp.where(M > 0.0, e, -1e30)
            emax = jnp.max(smat, axis=0, keepdims=True)
            emax_n = jnp.sum(M * emax, axis=1, keepdims=True)
            ee = jnp.exp(e - emax_n)
            denom = jnp.sum(M * ee, axis=0, keepdims=True)
            denom_n = jnp.sum(M * denom, axis=1, keepdims=True)
            alpha = ee / denom_n
            readout = _nn_dot(MT, alpha * feat)
            q_star = jnp.concatenate([q, readout], axis=1)
        gsp = _nn_dot(q_star, wsp_ref[...]) + bsp_ref[...]
        gs.append(_prelu(gsp, asp_ref[...]))
    r_graph = 0.5 * (gs[0] + gs[1])
    cat = jnp.concatenate([r_graph, gs[2]], axis=1)
    re = _prelu(_nn_dot(cat, wem_ref[...]) + bem_ref[...], aem_ref[...])
    for r_i in range(2):
        ind = jnp.broadcast_to(rinds_ref[r_i:r_i + 1, :], (B, 8))
        inp = jnp.concatenate([gs[r_i] - re, ind], axis=1)
        h1 = _prelu(_nn_dot(inp, w1_ref[...]) + b1_ref[...], a1_ref[...])
        h2 = _prelu(_nn_dot(h1, w2_ref[...]) + b2_ref[...], a2_ref[...])
        o_ref[r_i, :, :] = _nn_dot(h2, w3_ref[...]) + b3_ref[...]


def _proj_call(x2, xp, w, b):
    return pl.pallas_call(
        _proj_body,
        out_shape=jax.ShapeDtypeStruct((NT, HP), jnp.float32),
    )(x2, xp, w, b)


def _gru_call(aggp, h, wih, whh, bih, bhh, bconv):
    return pl.pallas_call(
        _gru_body,
        out_shape=jax.ShapeDtypeStruct((NT, HP), jnp.float32),
    )(aggp, h, wih, whh, bih, bhh, bconv)


def _s2s_call(h0, h, n2gc, n2gr, rinds, args):
    return pl.pallas_call(
        _s2s_body,
        out_shape=jax.ShapeDtypeStruct((2, B, 128), jnp.float32),
    )(h0, h, n2gc, n2gr, rinds, *args)


def kernel(r_node_feats, r_edge_feats, p_node_feats, p_edge_feats, rinds,
           params, r_edge_index, p_edge_index, node2graph):
    p = params
    f32 = jnp.float32

    ef = jnp.concatenate([r_edge_feats[0], r_edge_feats[1], p_edge_feats],
                         axis=0)
    src = jnp.concatenate([
        r_edge_index[0, 0], r_edge_index[1, 0] + N, p_edge_index[0] + 2 * N])
    dst = jnp.concatenate([
        r_edge_index[0, 1], r_edge_index[1, 1] + N, p_edge_index[1] + 2 * N])
    dst0 = jnp.where(dst < HN, dst, HN)
    dst1 = jnp.where(dst >= HN, dst - HN, HN)
    dstc = jnp.concatenate([
        jnp.pad(d.reshape(NS, NCH2, CH), ((0, 0), (0, 16 - NCH2), (0, 0)))
        for d in (dst0, dst1)], axis=0).reshape(NC * NS * 16, CH)
    zeros = jnp.zeros((HSLAB, HP), f32)

    bf16 = jnp.bfloat16
    rmat = jnp.kron(jnp.eye(16, dtype=f32), jnp.ones((1, H), f32))
    w2p = jnp.concatenate([
        p['W_edge'].reshape(16 * H, H),
        p['b_edge'].reshape(H, H)], axis=0)


    h0 = _proj_call(r_node_feats, p_node_feats,
                    p['W_proj'], p['b_proj'].reshape(1, H))

    def _step(_, h):
        hs = _sc_gather(h, src)
        msg = _msg_call(hs, ef, rmat, w2p)
        aggp = _sc_scatter(msg, dstc, zeros)
        return _gru_call(aggp, h, p['W_ih'], p['W_hh'],
                         p['b_ih'].reshape(1, 3 * H),
                         p['b_hh'].reshape(1, 3 * H),
                         p['b_conv'].reshape(1, H))

    h = lax.fori_loop(0, 3, _step, h0)

    s2s_args = (
        p['W_ih_s'], p['W_hh_s'],
        p['b_ih_s'].reshape(1, 8 * H), p['b_hh_s'].reshape(1, 8 * H),
        p['W_sp'], p['b_sp'].reshape(1, 1024), p['a_sp'].reshape(1, 1),
        p['W_em'], p['b_em'].reshape(1, 1024), p['a_em'].reshape(1, 1),
        p['W1'], p['b1'].reshape(1, 512), p['a1'].reshape(1, 1),
        p['W2'], p['b2'].reshape(1, 512), p['a2'].reshape(1, 1),
        jnp.pad(p['W3'], ((0, 0), (0, 126))),
        jnp.pad(p['b3'], (0, 126)).reshape(1, 128),
    )
    res = _s2s_call(h0, h, node2graph.reshape(N, 1), node2graph.reshape(1, N),
                    rinds, s2s_args)
    return jnp.transpose(res[:, :, 0:2], (1, 0, 2))
